# Initial kernel scaffold; baseline (speedup 1.0000x reference)
import jax
import jax.numpy as jnp
from jax.experimental import pallas as pl


def kernel(x, weight):
    raise NotImplementedError("write your pallas kernel here")


# speedup vs baseline: 5.5636x; 5.5636x over previous
import functools

import jax
import jax.numpy as jnp
from jax import lax
from jax.experimental import pallas as pl
from jax.experimental.pallas import tpu as pltpu
from jax.experimental.pallas import tpu_sc as plsc

VOCAB_SIZE = 1000
EMBED_DIM = 128
BATCH = 4096
SEQ = 200

NC = 2
NS = 16
NW = NC * NS

B = BATCH * SEQ
B_PER_W = B // NW
IDX_ROWS = B_PER_W // 128
G = 2
CHUNK = G * 128
NCHUNKS = IDX_ROWS // G


def _make_kernel():
    mesh = plsc.VectorSubcoreMesh(
        core_axis_name="c", subcore_axis_name="s",
        num_cores=NC, num_subcores=NS)

    @functools.partial(
        pl.kernel,
        mesh=mesh,
        out_type=jax.ShapeDtypeStruct((B, EMBED_DIM), jnp.float32),
        scratch_types=[
            pltpu.VMEM((IDX_ROWS, 128), jnp.int32),
            pltpu.VMEM((CHUNK, EMBED_DIM), jnp.float32),
            pltpu.SemaphoreType.DMA,
        ],
    )
    def emb_kernel(x_hbm, w_hbm, out_hbm, idx_v, rows_v, sem):
        wid = lax.axis_index("s") * NC + lax.axis_index("c")
        base = wid * B_PER_W
        pltpu.sync_copy(x_hbm.at[wid], idx_v)

        def chunk_body(i, carry):
            cps = []
            for g in range(G):
                cps.append(pltpu.async_copy(
                    w_hbm.at[idx_v.at[i * G + g]],
                    rows_v.at[pl.ds(g * 128, 128)],
                    sem))
            for cp in cps:
                cp.wait()
            pltpu.sync_copy(
                rows_v, out_hbm.at[pl.ds(base + i * CHUNK, CHUNK)])
            return carry

        lax.fori_loop(0, NCHUNKS, chunk_body, 0)

    return emb_kernel


_emb = _make_kernel()


def kernel(x, weight):
    x3 = x.reshape(NW, IDX_ROWS, 128)
    out = _emb(x3, weight)
    return out.reshape(BATCH, SEQ, EMBED_DIM)


# speedup vs baseline: 5.6282x; 1.0116x over previous
import functools

import jax
import jax.numpy as jnp
from jax import lax
from jax.experimental import pallas as pl
from jax.experimental.pallas import tpu as pltpu
from jax.experimental.pallas import tpu_sc as plsc

VOCAB_SIZE = 1000
EMBED_DIM = 128
BATCH = 4096
SEQ = 200

NC = 2
NS = 16
NW = NC * NS

B = BATCH * SEQ
B_PER_W = B // NW
ROWS = 128
NCH = B_PER_W // ROWS
NSLOT = 4


def _make_kernel():
    mesh = plsc.VectorSubcoreMesh(
        core_axis_name="c", subcore_axis_name="s",
        num_cores=NC, num_subcores=NS)

    @functools.partial(
        pl.kernel,
        mesh=mesh,
        out_type=jax.ShapeDtypeStruct((B, EMBED_DIM), jnp.float32),
        scratch_types=[
            pltpu.VMEM((NCH, ROWS), jnp.int32),
            pltpu.VMEM((NSLOT, ROWS, EMBED_DIM), jnp.float32),
            [pltpu.SemaphoreType.DMA] * NSLOT,
            pltpu.SemaphoreType.DMA,
        ],
    )
    def emb_kernel(x_hbm, w_hbm, out_hbm, idx_v, rows_v, gsems, osem):
        wid = lax.axis_index("s") * NC + lax.axis_index("c")
        base = wid * B_PER_W
        pltpu.sync_copy(x_hbm.at[wid], idx_v)

        def fire_gather(c, slot):
            return pltpu.async_copy(
                w_hbm.at[idx_v.at[c]], rows_v.at[slot], gsems[slot])

        def drain_gather(c, slot):
            pltpu.make_async_copy(
                w_hbm.at[idx_v.at[c]], rows_v.at[slot], gsems[slot]).wait()

        def consume(c, slot, fire_next):
            drain_gather(c, slot)
            cp = pltpu.async_copy(
                rows_v.at[slot], out_hbm.at[pl.ds(base + c * ROWS, ROWS)],
                osem)
            cp.wait()
            if fire_next:
                fire_gather(c + NSLOT - 1, (slot + NSLOT - 1) % NSLOT)

        for c in range(NSLOT - 1):
            fire_gather(c, c)

        def body(m, carry):
            for u in range(NSLOT):
                consume(m * NSLOT + u, u, fire_next=True)
            return carry

        n_main = (NCH - NSLOT) // NSLOT
        lax.fori_loop(0, n_main, body, 0)

        tail = n_main * NSLOT
        fire_gather(NCH - 1, (NCH - 1) % NSLOT)
        for u in range(NSLOT):
            consume(tail + u, u, fire_next=False)

    return emb_kernel


_emb = _make_kernel()


def kernel(x, weight):
    x3 = x.reshape(NW, NCH, ROWS)
    out = _emb(x3, weight)
    return out.reshape(BATCH, SEQ, EMBED_DIM)


# speedup vs baseline: 16.0580x; 2.8532x over previous
import functools

import jax
import jax.numpy as jnp
from jax import lax
from jax.experimental import pallas as pl
from jax.experimental.pallas import tpu as pltpu
from jax.experimental.pallas import tpu_sc as plsc

VOCAB_SIZE = 1000
EMBED_DIM = 128
BATCH = 4096
SEQ = 200

NC = 2
NS = 16
NW = NC * NS

B = BATCH * SEQ
B_PER_W = B // NW
ROWS = 128
NCH = B_PER_W // ROWS
NSLOT = 4


def _make_kernel():
    mesh = plsc.VectorSubcoreMesh(
        core_axis_name="c", subcore_axis_name="s",
        num_cores=NC, num_subcores=NS)

    @functools.partial(
        pl.kernel,
        mesh=mesh,
        out_type=jax.ShapeDtypeStruct((B, EMBED_DIM), jnp.float32),
        scratch_types=[
            pltpu.VMEM((NCH, ROWS), jnp.int32),
            pltpu.VMEM((NSLOT, ROWS, EMBED_DIM), jnp.float32),
            pltpu.VMEM_SHARED((VOCAB_SIZE, EMBED_DIM), jnp.float32),
            [pltpu.SemaphoreType.DMA] * NSLOT,
            pltpu.SemaphoreType.DMA,
        ],
    )
    def emb_kernel(x_hbm, w_hbm, out_hbm, idx_v, rows_v, w_sh, gsems, osem):
        sid = lax.axis_index("s")
        wid = sid * NC + lax.axis_index("c")
        base = wid * B_PER_W

        @pl.when(sid == 0)
        def _stage_table():
            pltpu.sync_copy(w_hbm, w_sh)

        pltpu.sync_copy(x_hbm.at[wid], idx_v)
        plsc.subcore_barrier()

        def fire_gather(c, slot):
            return pltpu.async_copy(
                w_sh.at[idx_v.at[c]], rows_v.at[slot], gsems[slot])

        def drain_gather(c, slot):
            pltpu.make_async_copy(
                w_sh.at[idx_v.at[c]], rows_v.at[slot], gsems[slot]).wait()

        def consume(c, slot, fire_next):
            drain_gather(c, slot)
            cp = pltpu.async_copy(
                rows_v.at[slot], out_hbm.at[pl.ds(base + c * ROWS, ROWS)],
                osem)
            cp.wait()
            if fire_next:
                fire_gather(c + NSLOT - 1, (slot + NSLOT - 1) % NSLOT)

        for c in range(NSLOT - 1):
            fire_gather(c, c)

        def body(m, carry):
            for u in range(NSLOT):
                consume(m * NSLOT + u, u, fire_next=True)
            return carry

        n_main = (NCH - NSLOT) // NSLOT
        lax.fori_loop(0, n_main, body, 0)

        tail = n_main * NSLOT
        fire_gather(NCH - 1, (NCH - 1) % NSLOT)
        for u in range(NSLOT):
            consume(tail + u, u, fire_next=False)

    return emb_kernel


_emb = _make_kernel()


def kernel(x, weight):
    x3 = x.reshape(NW, NCH, ROWS)
    out = _emb(x3, weight)
    return out.reshape(BATCH, SEQ, EMBED_DIM)
